# baseline (speedup 1.0000x reference)
import functools

import jax
import jax.numpy as jnp
from jax import lax
from jax.experimental import pallas as pl
from jax.experimental.pallas import tpu as pltpu
from jax.experimental.pallas import tpu_sc as plsc

_LANE = 16
_NC, _NS = 2, 16
_NW = _NC * _NS
_BT = 128
_DT = 8
_P = 65


@functools.lru_cache(maxsize=None)
def _build(batch, seq_len, emb_dim):
    bpw = batch // _NW
    n_btiles = batch // _BT
    n_dtiles = emb_dim // _DT
    n_chunks = seq_len
    n_groups = _BT // _LANE
    n_dgroups = emb_dim // _LANE

    mesh = plsc.VectorSubcoreMesh(core_axis_name="c", subcore_axis_name="s")

    @functools.partial(
        pl.kernel,
        out_type=jax.ShapeDtypeStruct(
            (seq_len, n_dtiles, n_btiles, _DT, _BT), jnp.float32),
        mesh=mesh,
        scratch_types=[
            pltpu.VMEM((bpw * seq_len,), jnp.int32),
            pltpu.VMEM((2, _BT), jnp.int32),
            pltpu.VMEM((2, _BT, emb_dim), jnp.float32),
            pltpu.VMEM((2, _BT, _P), jnp.float32),
            pltpu.VMEM((2, 1, n_dtiles, 1, _DT, _BT), jnp.float32),
            pltpu.VMEM((seq_len, emb_dim), jnp.float32),
        ]
        + [pltpu.SemaphoreType.DMA] * 4,
        compiler_params=pltpu.CompilerParams(
            use_tc_tiling_on_sc=False, needs_layout_passes=False),
    )
    def emb_kernel(x_hbm, tok_hbm, pos_hbm, out_hbm, idx_v, idxt_v, gath_v,
                   rs_v, tr_v, pos_v, *sems):
        sem_g = sems[:2]
        sem_o = sems[2:]
        wid = lax.axis_index("s") * _NC + lax.axis_index("c")
        base = wid * bpw * seq_len
        pltpu.sync_copy(pos_hbm.at[pl.ds(0, seq_len)], pos_v)
        pltpu.sync_copy(x_hbm.at[pl.ds(base, bpw * seq_len)], idx_v)

        iota = lax.iota(jnp.int32, _LANE)
        iota_l = iota * seq_len

        def build_idxt(c, slot):
            for g in range(n_groups):
                inds = iota_l + (g * _LANE * seq_len + c)
                idxt_v[slot, pl.ds(g * _LANE, _LANE)] = (
                    plsc.load_gather(idx_v, [inds]))

        def fire_gather(c, slot):
            pltpu.async_copy(
                tok_hbm.at[idxt_v.at[slot]],
                gath_v.at[slot],
                sem_g[slot],
            )

        def wait_gather(c, slot):
            pltpu.make_async_copy(
                tok_hbm.at[idxt_v.at[slot]],
                gath_v.at[slot],
                sem_g[slot],
            ).wait()

        def fire_scatter(c, slot):
            pltpu.async_copy(
                tr_v.at[slot],
                out_hbm.at[pl.ds(c, 1), :, pl.ds(wid, 1)],
                sem_o[slot],
            )

        def wait_scatter(c, slot):
            pltpu.make_async_copy(
                tr_v.at[slot],
                out_hbm.at[pl.ds(c, 1), :, pl.ds(wid, 1)],
                sem_o[slot],
            ).wait()

        def compute(c, slot):
            l = c
            pvs = [pos_v[l, pl.ds(k * _LANE, _LANE)] for k in range(n_dgroups)]

            def rs_body(b8, _):
                for r in range(8):
                    bl = b8 * 8 + r
                    for k in range(n_dgroups):
                        rs_v[slot, bl, pl.ds(k * _LANE, _LANE)] = (
                            gath_v[slot, bl, pl.ds(k * _LANE, _LANE)]
                            + pvs[k])
                return 0

            lax.fori_loop(0, _BT // 8, rs_body, 0)

            def dt_body(dt, _):
                for di in range(_DT):
                    d = dt * _DT + di
                    dv = jnp.broadcast_to(d, (_LANE,))
                    for g in range(n_groups):
                        blv = iota + (g * _LANE)
                        v = plsc.load_gather(rs_v.at[slot], [blv, dv])
                        tr_v[slot, 0, dt, 0, di, pl.ds(g * _LANE, _LANE)] = v
                return 0

            lax.fori_loop(0, n_dtiles, dt_body, 0)

        build_idxt(0, 0)
        fire_gather(0, 0)
        build_idxt(1, 1)
        fire_gather(1, 1)

        def body(q, carry):
            for j in range(2):
                c = q * 2 + j
                wait_gather(c, j)

                @pl.when(c >= 2)
                def _():
                    wait_scatter(c - 2, j)

                compute(c, j)
                fire_scatter(c, j)

                @pl.when(c + 2 < n_chunks)
                def _():
                    build_idxt(c + 2, j)
                    fire_gather(c + 2, j)
            return carry

        lax.fori_loop(0, n_chunks // 2, body, 0)
        wait_scatter(n_chunks - 2, 0)
        wait_scatter(n_chunks - 1, 1)

    return emb_kernel


@jax.jit
def kernel(x, token_table, pos_table):
    batch, seq_len = x.shape
    emb_dim = token_table.shape[1]
    xf = x.reshape(-1).astype(jnp.int32)
    out5 = _build(batch, seq_len, emb_dim)(xf, token_table, pos_table)
    return out5.transpose(2, 4, 0, 1, 3).reshape(batch, seq_len, emb_dim)


# speedup vs baseline: 1.5710x; 1.5710x over previous
import functools

import jax
import jax.numpy as jnp
from jax import lax
from jax.experimental import pallas as pl
from jax.experimental.pallas import tpu as pltpu
from jax.experimental.pallas import tpu_sc as plsc

_LANE = 16
_NC, _NS = 2, 16
_NW = _NC * _NS
_NSLOTS = 4


@functools.lru_cache(maxsize=None)
def _build(batch, seq_len, emb_dim):
    n_tok = batch * seq_len
    tok_per_w = n_tok // _NW
    tok_chunk = seq_len
    rows_per_w = batch // _NW
    n_chunks = tok_per_w // tok_chunk
    n_groups = emb_dim // _LANE
    subs = []
    off = 0
    while off < tok_chunk:
        sz = min(128, tok_chunk - off)
        subs.append((off, sz))
        off += sz

    mesh = plsc.VectorSubcoreMesh(core_axis_name="c", subcore_axis_name="s")

    @functools.partial(
        pl.kernel,
        out_type=jax.ShapeDtypeStruct((batch, seq_len, emb_dim), jnp.float32),
        mesh=mesh,
        scratch_types=[
            pltpu.VMEM((tok_per_w,), jnp.int32),
            pltpu.VMEM((_NSLOTS, tok_chunk, emb_dim), jnp.float32),
            pltpu.VMEM((seq_len, emb_dim), jnp.float32),
        ]
        + [pltpu.SemaphoreType.DMA] * (2 * _NSLOTS),
        compiler_params=pltpu.CompilerParams(use_tc_tiling_on_sc=False),
    )
    def emb_kernel(x_hbm, tok_hbm, pos_hbm, out_hbm, idx_v, rows_v, pos_v,
                   *sems):
        sem_g = sems[:_NSLOTS]
        sem_o = sems[_NSLOTS:]
        wid = lax.axis_index("s") * _NC + lax.axis_index("c")
        base = wid * tok_per_w
        brow0 = wid * rows_per_w
        pltpu.sync_copy(pos_hbm.at[pl.ds(0, seq_len)], pos_v)
        pltpu.sync_copy(x_hbm.at[pl.ds(base, tok_per_w)], idx_v)

        def fire_gather(c, slot):
            for so, sz in subs:
                pltpu.async_copy(
                    tok_hbm.at[idx_v.at[pl.ds(c * tok_chunk + so, sz)]],
                    rows_v.at[slot].at[pl.ds(so, sz)],
                    sem_g[slot],
                )

        def wait_gather(c, slot):
            for so, sz in subs:
                pltpu.make_async_copy(
                    tok_hbm.at[idx_v.at[pl.ds(c * tok_chunk + so, sz)]],
                    rows_v.at[slot].at[pl.ds(so, sz)],
                    sem_g[slot],
                ).wait()

        def fire_scatter(c, slot):
            pltpu.async_copy(
                rows_v.at[slot],
                out_hbm.at[brow0 + c],
                sem_o[slot],
            )

        def wait_scatter(c, slot):
            pltpu.make_async_copy(
                rows_v.at[slot],
                out_hbm.at[brow0 + c],
                sem_o[slot],
            ).wait()

        fire_gather(0, 0)
        fire_gather(1, 1)

        def body(q, carry):
            for j in range(_NSLOTS):
                c = q * _NSLOTS + j
                nxt = (j + 2) % _NSLOTS

                @pl.when(c + 2 < n_chunks)
                def _():
                    @pl.when(c >= 2)
                    def _():
                        wait_scatter(c - 2, nxt)

                    fire_gather(c + 2, nxt)

                wait_gather(c, j)

                def add_body(u, _):
                    for r in range(2):
                        t = u * 2 + r
                        for d in range(n_groups):
                            sl = pl.ds(d * _LANE, _LANE)
                            rows_v[j, t, sl] += pos_v[t, sl]
                    return 0

                lax.fori_loop(0, tok_chunk // 2, add_body, 0)
                fire_scatter(c, j)
            return carry

        lax.fori_loop(0, n_chunks // _NSLOTS, body, 0)
        wait_scatter(n_chunks - 2, (n_chunks - 2) % _NSLOTS)
        wait_scatter(n_chunks - 1, (n_chunks - 1) % _NSLOTS)

    return emb_kernel


@jax.jit
def kernel(x, token_table, pos_table):
    batch, seq_len = x.shape
    emb_dim = token_table.shape[1]
    xf = x.reshape(-1).astype(jnp.int32)
    return _build(batch, seq_len, emb_dim)(xf, token_table, pos_table)
